# baseline (speedup 1.0000x reference)
import functools

import jax
import jax.numpy as jnp
from jax.experimental import pallas as pl
from jax.experimental.pallas import tpu as pltpu

B, S, D, E = 2, 2048, 1024, 8
T = B * S
EP = 128
M = 512
NB = T // M


def _gate_body(x_ref, wg_ref, gate_ref, selp_ref, psum_ref, cnt_ref):
    xb = x_ref[...]
    wg = wg_ref[...]
    logits = jnp.dot(xb, wg, preferred_element_type=jnp.float32)
    lane = jax.lax.broadcasted_iota(jnp.int32, (M, EP), 1)
    valid = lane < E
    neg = jnp.full_like(logits, -jnp.inf)
    logit_m = jnp.where(valid, logits, neg)
    mx = jnp.max(logit_m, axis=-1, keepdims=True)
    ex = jnp.where(valid, jnp.exp(logit_m - mx), 0.0)
    den = jnp.sum(ex, axis=-1, keepdims=True)
    probs = ex / den
    gate = jnp.argmax(logit_m, axis=-1).astype(jnp.int32)
    selp = jnp.max(probs, axis=-1)
    gate_ref[0, 0, :] = gate
    selp_ref[0, 0, :] = selp
    psum_ref[0, 0, :] = jnp.sum(probs, axis=0)
    onehot = jnp.where(lane == gate[:, None], 1.0, 0.0)
    cnt_ref[0, 0, :] = jnp.sum(onehot, axis=0)


def _expert_body(gate_ref, selp_ref, x_ref, w_ref, b_ref, out_ref):
    e = pl.program_id(1)
    xb = x_ref[...]
    w = w_ref[0]
    y = jnp.dot(xb, w, preferred_element_type=jnp.float32) + b_ref[0, 0, :][None, :]
    gate = gate_ref[0, 0, :]
    selp = selp_ref[0, 0, :]
    scale = jnp.where(gate == e, selp, 0.0)
    contrib = y * scale[:, None]
    @pl.when(e == 0)
    def _():
        out_ref[...] = contrib
    @pl.when(e > 0)
    def _():
        out_ref[...] += contrib


def kernel(x, attention_mask, W_gate, W_experts, b_experts):
    del attention_mask
    xf = x.reshape(T, D)
    wg_pad = jnp.zeros((D, EP), jnp.float32).at[:, :E].set(W_gate)

    gate3, selp3, psum3, cnt3 = pl.pallas_call(
        _gate_body,
        grid=(NB,),
        in_specs=[
            pl.BlockSpec((M, D), lambda b: (b, 0)),
            pl.BlockSpec((D, EP), lambda b: (0, 0)),
        ],
        out_specs=[
            pl.BlockSpec((1, 1, M), lambda b: (b, 0, 0)),
            pl.BlockSpec((1, 1, M), lambda b: (b, 0, 0)),
            pl.BlockSpec((1, 1, EP), lambda b: (b, 0, 0)),
            pl.BlockSpec((1, 1, EP), lambda b: (b, 0, 0)),
        ],
        out_shape=[
            jax.ShapeDtypeStruct((NB, 1, M), jnp.int32),
            jax.ShapeDtypeStruct((NB, 1, M), jnp.float32),
            jax.ShapeDtypeStruct((NB, 1, EP), jnp.float32),
            jax.ShapeDtypeStruct((NB, 1, EP), jnp.float32),
        ],
    )(xf, wg_pad)

    out = pl.pallas_call(
        _expert_body,
        grid=(NB, E),
        in_specs=[
            pl.BlockSpec((1, 1, M), lambda b, e: (b, 0, 0)),
            pl.BlockSpec((1, 1, M), lambda b, e: (b, 0, 0)),
            pl.BlockSpec((M, D), lambda b, e: (b, 0)),
            pl.BlockSpec((1, D, D), lambda b, e: (e, 0, 0)),
            pl.BlockSpec((1, 1, D), lambda b, e: (e, 0, 0)),
        ],
        out_specs=pl.BlockSpec((M, D), lambda b, e: (b, 0)),
        out_shape=jax.ShapeDtypeStruct((T, D), jnp.float32),
    )(gate3, selp3, xf, W_experts, b_experts.reshape(E, 1, D))

    psum = jnp.sum(psum3[:, 0, :E], axis=0)
    counts_f = jnp.sum(cnt3[:, 0, :E], axis=0)
    P = psum / jnp.float32(T)
    f = counts_f / jnp.sum(counts_f)
    balance_loss = jnp.float32(E) * jnp.sum(P * f)
    gate_load = counts_f.astype(jnp.int32)
    return out.reshape(B, S, D), balance_loss, gate_load
